# baseline (speedup 1.0000x reference)
import jax
import jax.numpy as jnp
from jax.experimental import pallas as pl
from jax.experimental.pallas import tpu as pltpu

_K = 1024
_D = 64
_B = 16
_H = 32
_W = 32
_HW = _H * _W
_N = _B * _HW
_PB = 2
_T = _PB * _HW
_STEPS = _B // _PB
_COMMIT = 0.25


def _vq_body(z_ref, e_ref, enc_ref, q_ref, loss_ref, perp_ref, cnt_ref, sq_ref):
    s = pl.program_id(0)
    zt = jnp.concatenate([z_ref[i].T for i in range(_PB)], axis=0)
    e = e_ref[...]
    p = jax.lax.dot_general(zt, e, (((1,), (1,)), ((), ())),
                            preferred_element_type=jnp.float32)
    sz = jnp.sum(zt * zt, axis=1, keepdims=True)
    se = jnp.sum(e * e, axis=1)
    dist = (sz + se[None, :]) - 2.0 * p
    dmin = jnp.min(dist, axis=1, keepdims=True)
    kio = jax.lax.broadcasted_iota(jnp.int32, (_T, _K), 1)
    idx = jnp.min(jnp.where(dist == dmin, kio, _K), axis=1, keepdims=True)
    onehot = (kio == idx).astype(jnp.float32)
    enc_ref[...] = onehot
    q = jax.lax.dot_general(onehot, e, (((1,), (0,)), ((), ())),
                            preferred_element_type=jnp.float32)
    diff = q - zt
    qst = zt + diff
    for i in range(_PB):
        q_ref[i] = qst[i * _HW:(i + 1) * _HW].T
    bc = jnp.sum(onehot, axis=0, keepdims=True)
    bs = jnp.sum(jnp.sum(diff * diff, axis=1, keepdims=True),
                 axis=0, keepdims=True)

    @pl.when(s == 0)
    def _():
        cnt_ref[...] = bc
        sq_ref[...] = bs

    @pl.when(s > 0)
    def _():
        cnt_ref[...] += bc
        sq_ref[...] += bs

    @pl.when(s == _STEPS - 1)
    def _():
        avg = cnt_ref[...] * (1.0 / _N)
        ent = jnp.sum(avg * jnp.log(avg + 1e-10), axis=1, keepdims=True)
        perp_ref[...] = jnp.exp(-ent)
        loss_ref[...] = sq_ref[...] * (_COMMIT / (_N * _D))


def kernel(z_e, embedding_weight):
    z3 = z_e.reshape(_B, _D, _HW)
    enc, q3, loss, perp = pl.pallas_call(
        _vq_body,
        grid=(_STEPS,),
        in_specs=[pl.BlockSpec((_PB, _D, _HW), lambda s: (s, 0, 0)),
                  pl.BlockSpec((_K, _D), lambda s: (0, 0))],
        out_specs=[pl.BlockSpec((_T, _K), lambda s: (s, 0)),
                   pl.BlockSpec((_PB, _D, _HW), lambda s: (s, 0, 0)),
                   pl.BlockSpec((1, 1), lambda s: (0, 0)),
                   pl.BlockSpec((1, 1), lambda s: (0, 0))],
        out_shape=[jax.ShapeDtypeStruct((_N, _K), jnp.float32),
                   jax.ShapeDtypeStruct((_B, _D, _HW), jnp.float32),
                   jax.ShapeDtypeStruct((1, 1), jnp.float32),
                   jax.ShapeDtypeStruct((1, 1), jnp.float32)],
        scratch_shapes=[pltpu.VMEM((1, _K), jnp.float32),
                        pltpu.VMEM((1, 1), jnp.float32)],
        compiler_params=pltpu.CompilerParams(
            dimension_semantics=("arbitrary",)),
    )(z3, embedding_weight)
    q_out = q3.reshape(_B, _D, _H, _W)
    return (q_out, loss[0, 0], perp[0, 0], enc)
